# Initial kernel scaffold; baseline (speedup 1.0000x reference)
import jax
import jax.numpy as jnp
from jax.experimental import pallas as pl


def kernel(X_train, ts, x_init, zs, W):
    raise NotImplementedError("write your pallas kernel here")


# speedup vs baseline: 4.5772x; 4.5772x over previous
import jax
import jax.numpy as jnp
from jax.experimental import pallas as pl
from jax.experimental.pallas import tpu as pltpu

_Q = 1024
_K = 100000
_D = 32
_T = 10
_THRESHOLD = 1.0 / 3.0
_KB = 2048
_NK = (_K + _KB - 1) // _KB
_BIG = 3.0e38


def _knn_kernel(ts_ref, xinit_ref, zs_ref, w_ref, xt_ref, out_ref,
                xs_scr, m1_scr, m2_scr):
    pid = pl.program_id(0)

    @pl.when(pid == 0)
    def _prologue():
        x = xinit_ref[...]
        w = w_ref[...]
        for i in range(_T - 1):
            dt = ts_ref[i + 1] - ts_ref[i]
            x = (x + jnp.dot(x, w, preferred_element_type=jnp.float32) * dt
                 + jnp.sqrt(jnp.abs(dt)) * zs_ref[:, i, :])
        xs_scr[...] = x
        m1_scr[...] = jnp.full((_Q, 1), _BIG, jnp.float32)
        m2_scr[...] = jnp.full((_Q, 1), _BIG, jnp.float32)

    x = xs_scr[...]
    xt = xt_ref[...]
    xs2 = jnp.sum(x * x, axis=1, keepdims=True)
    xt2 = jnp.sum(xt * xt, axis=1)
    prod = jax.lax.dot_general(x, xt, (((1,), (1,)), ((), ())),
                               preferred_element_type=jnp.float32)
    d2 = jnp.maximum(xs2 + xt2[None, :] - 2.0 * prod, 0.0)
    col = jax.lax.broadcasted_iota(jnp.int32, (1, _KB), 1) + pid * _KB
    d2 = jnp.where(col < _K, d2, _BIG)

    b1 = jnp.min(d2, axis=1, keepdims=True)
    eq = d2 == b1
    cnt = jnp.sum(eq.astype(jnp.float32), axis=1, keepdims=True)
    b2m = jnp.min(jnp.where(eq, _BIG, d2), axis=1, keepdims=True)
    b2 = jnp.where(cnt > 1.0, b1, b2m)

    m1 = m1_scr[...]
    m2 = m2_scr[...]
    m1_scr[...] = jnp.minimum(m1, b1)
    m2_scr[...] = jnp.minimum(jnp.maximum(m1, b1), jnp.minimum(m2, b2))

    @pl.when(pid == _NK - 1)
    def _epilogue():
        d0 = jnp.sqrt(m1_scr[...])
        d1 = jnp.sqrt(m2_scr[...])
        mem = (d0 < _THRESHOLD * d1).astype(jnp.float32)
        out_ref[...] = jnp.sum(mem, keepdims=True).reshape(1, 1) / jnp.float32(_Q)


def kernel(X_train, ts, x_init, zs, W):
    out = pl.pallas_call(
        _knn_kernel,
        grid=(_NK,),
        in_specs=[
            pl.BlockSpec(memory_space=pltpu.SMEM),
            pl.BlockSpec((_Q, _D), lambda k: (0, 0)),
            pl.BlockSpec((_Q, _T - 1, _D), lambda k: (0, 0, 0)),
            pl.BlockSpec((_D, _D), lambda k: (0, 0)),
            pl.BlockSpec((_KB, _D), lambda k: (k, 0)),
        ],
        out_specs=pl.BlockSpec((1, 1), lambda k: (0, 0)),
        out_shape=jax.ShapeDtypeStruct((1, 1), jnp.float32),
        scratch_shapes=[
            pltpu.VMEM((_Q, _D), jnp.float32),
            pltpu.VMEM((_Q, 1), jnp.float32),
            pltpu.VMEM((_Q, 1), jnp.float32),
        ],
        compiler_params=pltpu.CompilerParams(
            dimension_semantics=("arbitrary",),
        ),
    )(ts, x_init, zs, W, X_train)
    return out[0, 0]


# speedup vs baseline: 5.1274x; 1.1202x over previous
import jax
import jax.numpy as jnp
from jax.experimental import pallas as pl
from jax.experimental.pallas import tpu as pltpu

_Q = 1024
_K = 100000
_D = 32
_T = 10
_THRESHOLD = 1.0 / 3.0
_KB = 2048
_NK = (_K + _KB - 1) // _KB
_KPAD = _NK * _KB
_BIG = 3.0e38
_FAR = 1.0e17


def _knn_kernel(ts_ref, xinit_ref, zs_ref, w_ref, xt_ref, out_ref,
                xa_scr, xs2_scr, m1_scr, m2_scr):
    pid = pl.program_id(0)

    @pl.when(pid == 0)
    def _prologue():
        x = xinit_ref[...]
        w = w_ref[...]
        for i in range(_T - 1):
            dt = ts_ref[i + 1] - ts_ref[i]
            x = (x + jnp.dot(x, w, preferred_element_type=jnp.float32) * dt
                 + jnp.sqrt(jnp.abs(dt)) * zs_ref[:, i, :])
        xs2_scr[...] = jnp.sum(x * x, axis=1, keepdims=True)
        xa_scr[...] = jnp.concatenate(
            [-2.0 * x, jnp.ones((_Q, 1), jnp.float32)], axis=1)
        m1_scr[...] = jnp.full((_Q, 1), _BIG, jnp.float32)
        m2_scr[...] = jnp.full((_Q, 1), _BIG, jnp.float32)

    xa = xa_scr[...]
    xt = xt_ref[...]
    xt2 = jnp.sum(xt * xt, axis=1, keepdims=True)
    xta = jnp.concatenate([xt, xt2], axis=1)
    e = jax.lax.dot_general(xa, xta, (((1,), (1,)), ((), ())),
                            preferred_element_type=jnp.float32)

    b1 = jnp.min(e, axis=1, keepdims=True)
    eq = e == b1
    cnt = jnp.sum(eq.astype(jnp.float32), axis=1, keepdims=True)
    b2m = jnp.min(jnp.where(eq, _BIG, e), axis=1, keepdims=True)
    b2 = jnp.where(cnt > 1.0, b1, b2m)

    m1 = m1_scr[...]
    m2 = m2_scr[...]
    m1_scr[...] = jnp.minimum(m1, b1)
    m2_scr[...] = jnp.minimum(jnp.maximum(m1, b1), jnp.minimum(m2, b2))

    @pl.when(pid == _NK - 1)
    def _epilogue():
        xs2 = xs2_scr[...]
        d0 = jnp.sqrt(jnp.maximum(m1_scr[...] + xs2, 0.0))
        d1 = jnp.sqrt(jnp.maximum(m2_scr[...] + xs2, 0.0))
        mem = (d0 < _THRESHOLD * d1).astype(jnp.float32)
        out_ref[...] = jnp.sum(mem, keepdims=True).reshape(1, 1) / jnp.float32(_Q)


def kernel(X_train, ts, x_init, zs, W):
    xt_pad = jnp.pad(X_train, ((0, _KPAD - _K), (0, 0)),
                     constant_values=_FAR)
    out = pl.pallas_call(
        _knn_kernel,
        grid=(_NK,),
        in_specs=[
            pl.BlockSpec(memory_space=pltpu.SMEM),
            pl.BlockSpec((_Q, _D), lambda k: (0, 0)),
            pl.BlockSpec((_Q, _T - 1, _D), lambda k: (0, 0, 0)),
            pl.BlockSpec((_D, _D), lambda k: (0, 0)),
            pl.BlockSpec((_KB, _D), lambda k: (k, 0)),
        ],
        out_specs=pl.BlockSpec((1, 1), lambda k: (0, 0)),
        out_shape=jax.ShapeDtypeStruct((1, 1), jnp.float32),
        scratch_shapes=[
            pltpu.VMEM((_Q, _D + 1), jnp.float32),
            pltpu.VMEM((_Q, 1), jnp.float32),
            pltpu.VMEM((_Q, 1), jnp.float32),
            pltpu.VMEM((_Q, 1), jnp.float32),
        ],
        compiler_params=pltpu.CompilerParams(
            dimension_semantics=("arbitrary",),
        ),
    )(ts, x_init, zs, W, xt_pad)
    return out[0, 0]


# speedup vs baseline: 7.7437x; 1.5102x over previous
import jax
import jax.numpy as jnp
from jax.experimental import pallas as pl
from jax.experimental.pallas import tpu as pltpu

_Q = 1024
_K = 100000
_D = 32
_T = 10
_THRESHOLD = 1.0 / 3.0
_KB = 2048
_NK = (_K + _KB - 1) // _KB
_KPAD = _NK * _KB
_BIG = 3.0e38
_FAR = 1.0e17
_L = 128


def _knn_kernel(ts_ref, xinit_ref, zs_ref, w_ref, xt_ref, out_ref,
                xa_scr, xs2_scr, m1_scr, m2_scr):
    pid = pl.program_id(0)

    @pl.when(pid == 0)
    def _prologue():
        x = xinit_ref[...]
        w = w_ref[...]
        for i in range(_T - 1):
            dt = ts_ref[i + 1] - ts_ref[i]
            x = (x + jnp.dot(x, w, preferred_element_type=jnp.float32) * dt
                 + jnp.sqrt(jnp.abs(dt)) * zs_ref[i])
        xs2_scr[...] = jnp.sum(x * x, axis=1, keepdims=True)
        xa_scr[...] = jnp.concatenate(
            [-2.0 * x, jnp.ones((_Q, 1), jnp.float32)], axis=1)
        m1_scr[...] = jnp.full((_Q, _L), _BIG, jnp.float32)
        m2_scr[...] = jnp.full((_Q, _L), _BIG, jnp.float32)

    xa = xa_scr[...]
    xt = xt_ref[...]
    xt2 = jnp.sum(xt * xt, axis=1, keepdims=True)
    xta = jnp.concatenate([xt, xt2], axis=1)
    e = jax.lax.dot_general(xa, xta, (((1,), (1,)), ((), ())),
                            preferred_element_type=jnp.float32)

    m1 = m1_scr[...]
    m2 = m2_scr[...]
    for j in range(_KB // _L):
        v = e[:, j * _L:(j + 1) * _L]
        hi = jnp.maximum(m1, v)
        m1 = jnp.minimum(m1, v)
        m2 = jnp.minimum(m2, hi)
    m1_scr[...] = m1
    m2_scr[...] = m2

    @pl.when(pid == _NK - 1)
    def _epilogue():
        M1 = m1_scr[...]
        M2 = m2_scr[...]
        b1 = jnp.min(M1, axis=1, keepdims=True)
        eq = M1 == b1
        cnt = jnp.sum(eq.astype(jnp.float32), axis=1, keepdims=True)
        c_m1 = jnp.min(jnp.where(eq, _BIG, M1), axis=1, keepdims=True)
        c_m2 = jnp.min(jnp.where(eq, M2, _BIG), axis=1, keepdims=True)
        b2 = jnp.where(cnt > 1.0, b1, jnp.minimum(c_m1, c_m2))
        xs2 = xs2_scr[...]
        d0 = jnp.sqrt(jnp.maximum(b1 + xs2, 0.0))
        d1 = jnp.sqrt(jnp.maximum(b2 + xs2, 0.0))
        mem = (d0 < _THRESHOLD * d1).astype(jnp.float32)
        out_ref[...] = jnp.sum(mem, keepdims=True).reshape(1, 1) / jnp.float32(_Q)


def kernel(X_train, ts, x_init, zs, W):
    xt_pad = jnp.pad(X_train, ((0, _KPAD - _K), (0, 0)),
                     constant_values=_FAR)
    zs_t = jnp.transpose(zs, (1, 0, 2))
    out = pl.pallas_call(
        _knn_kernel,
        grid=(_NK,),
        in_specs=[
            pl.BlockSpec(memory_space=pltpu.SMEM),
            pl.BlockSpec((_Q, _D), lambda k: (0, 0)),
            pl.BlockSpec((_T - 1, _Q, _D), lambda k: (0, 0, 0)),
            pl.BlockSpec((_D, _D), lambda k: (0, 0)),
            pl.BlockSpec((_KB, _D), lambda k: (k, 0)),
        ],
        out_specs=pl.BlockSpec((1, 1), lambda k: (0, 0)),
        out_shape=jax.ShapeDtypeStruct((1, 1), jnp.float32),
        scratch_shapes=[
            pltpu.VMEM((_Q, _D + 1), jnp.float32),
            pltpu.VMEM((_Q, 1), jnp.float32),
            pltpu.VMEM((_Q, _L), jnp.float32),
            pltpu.VMEM((_Q, _L), jnp.float32),
        ],
        compiler_params=pltpu.CompilerParams(
            dimension_semantics=("arbitrary",),
        ),
    )(ts, x_init, zs_t, W, xt_pad)
    return out[0, 0]


# speedup vs baseline: 8.6355x; 1.1152x over previous
import jax
import jax.numpy as jnp
from jax.experimental import pallas as pl
from jax.experimental.pallas import tpu as pltpu

_Q = 1024
_K = 100000
_D = 32
_T = 10
_THRESHOLD = 1.0 / 3.0
_KB = 2048
_NK = (_K + _KB - 1) // _KB
_KPAD = _NK * _KB
_BIG = 3.0e38
_FAR = 1.0e17
_L = 128


def _knn_kernel(ts_ref, xinit_ref, zs_ref, w_ref, xt_ref, out_ref,
                xa_scr, xs2_scr, m1_scr, m2_scr):
    pid = pl.program_id(0)

    @pl.when(pid == 0)
    def _prologue():
        x = xinit_ref[...]
        w = w_ref[...]
        for i in range(_T - 1):
            dt = ts_ref[i + 1] - ts_ref[i]
            x = (x + jnp.dot(x, w, preferred_element_type=jnp.float32) * dt
                 + jnp.sqrt(jnp.abs(dt)) * zs_ref[i])
        xs2_scr[...] = jnp.sum(x * x, axis=1, keepdims=True)
        xa_scr[...] = jnp.concatenate(
            [-2.0 * x, jnp.ones((_Q, 1), jnp.float32)],
            axis=1).astype(jnp.bfloat16)
        m1_scr[...] = jnp.full((_Q, _L), _BIG, jnp.float32)
        m2_scr[...] = jnp.full((_Q, _L), _BIG, jnp.float32)

    xa = xa_scr[...]
    xt = xt_ref[...]
    xtf = xt.astype(jnp.float32)
    xt2 = jnp.sum(xtf * xtf, axis=1, keepdims=True).astype(jnp.bfloat16)
    xta = jnp.concatenate([xt, xt2], axis=1)

    m1 = m1_scr[...]
    m2 = m2_scr[...]
    for j in range(_KB // _L):
        v = jax.lax.dot_general(xa, xta[j * _L:(j + 1) * _L, :],
                                (((1,), (1,)), ((), ())),
                                preferred_element_type=jnp.float32)
        hi = jnp.maximum(m1, v)
        m1 = jnp.minimum(m1, v)
        m2 = jnp.minimum(m2, hi)
    m1_scr[...] = m1
    m2_scr[...] = m2

    @pl.when(pid == _NK - 1)
    def _epilogue():
        M1 = m1_scr[...]
        M2 = m2_scr[...]
        b1 = jnp.min(M1, axis=1, keepdims=True)
        eq = M1 == b1
        cnt = jnp.sum(eq.astype(jnp.float32), axis=1, keepdims=True)
        c_m1 = jnp.min(jnp.where(eq, _BIG, M1), axis=1, keepdims=True)
        c_m2 = jnp.min(jnp.where(eq, M2, _BIG), axis=1, keepdims=True)
        b2 = jnp.where(cnt > 1.0, b1, jnp.minimum(c_m1, c_m2))
        xs2 = xs2_scr[...]
        d0 = jnp.sqrt(jnp.maximum(b1 + xs2, 0.0))
        d1 = jnp.sqrt(jnp.maximum(b2 + xs2, 0.0))
        mem = (d0 < _THRESHOLD * d1).astype(jnp.float32)
        out_ref[...] = jnp.sum(mem, keepdims=True).reshape(1, 1) / jnp.float32(_Q)


def kernel(X_train, ts, x_init, zs, W):
    xt_pad = jnp.pad(X_train, ((0, _KPAD - _K), (0, 0)),
                     constant_values=_FAR).astype(jnp.bfloat16)
    zs_t = jnp.transpose(zs, (1, 0, 2))
    out = pl.pallas_call(
        _knn_kernel,
        grid=(_NK,),
        in_specs=[
            pl.BlockSpec(memory_space=pltpu.SMEM),
            pl.BlockSpec((_Q, _D), lambda k: (0, 0)),
            pl.BlockSpec((_T - 1, _Q, _D), lambda k: (0, 0, 0)),
            pl.BlockSpec((_D, _D), lambda k: (0, 0)),
            pl.BlockSpec((_KB, _D), lambda k: (k, 0)),
        ],
        out_specs=pl.BlockSpec((1, 1), lambda k: (0, 0)),
        out_shape=jax.ShapeDtypeStruct((1, 1), jnp.float32),
        scratch_shapes=[
            pltpu.VMEM((_Q, _D + 1), jnp.bfloat16),
            pltpu.VMEM((_Q, 1), jnp.float32),
            pltpu.VMEM((_Q, _L), jnp.float32),
            pltpu.VMEM((_Q, _L), jnp.float32),
        ],
        compiler_params=pltpu.CompilerParams(
            dimension_semantics=("arbitrary",),
        ),
    )(ts, x_init, zs_t, W, xt_pad)
    return out[0, 0]


# speedup vs baseline: 8.8757x; 1.0278x over previous
import jax
import jax.numpy as jnp
from jax.experimental import pallas as pl
from jax.experimental.pallas import tpu as pltpu

_Q = 1024
_K = 100000
_D = 32
_T = 10
_THRESHOLD = 1.0 / 3.0
_KB = 2048
_NK = (_K + _KB - 1) // _KB
_BIG = 3.0e38
_PADV = 1.0e30
_L = 128


def _knn_kernel(ts_ref, xinit_ref, zs_ref, w_ref, xt_ref, out_ref,
                xa_scr, xs2_scr, m1_scr, m2_scr):
    pid = pl.program_id(0)

    @pl.when(pid == 0)
    def _prologue():
        x = xinit_ref[...]
        w = w_ref[...]
        for i in range(_T - 1):
            dt = ts_ref[i + 1] - ts_ref[i]
            x = (x + jnp.dot(x, w, preferred_element_type=jnp.float32) * dt
                 + jnp.sqrt(jnp.abs(dt)) * zs_ref[:, i, :])
        xs2_scr[...] = jnp.sum(x * x, axis=1, keepdims=True)
        xa_scr[...] = jnp.concatenate(
            [-2.0 * x, jnp.ones((_Q, 1), jnp.float32)],
            axis=1).astype(jnp.bfloat16)
        m1_scr[...] = jnp.full((_Q, _L), _BIG, jnp.float32)
        m2_scr[...] = jnp.full((_Q, _L), _BIG, jnp.float32)

    xa = xa_scr[...]
    row = jax.lax.broadcasted_iota(jnp.int32, (_KB, 1), 0) + pid * _KB
    rmask = row < _K
    xtf = jnp.where(rmask, xt_ref[...], 0.0)
    xt2f = jnp.where(rmask,
                     jnp.sum(xtf * xtf, axis=1, keepdims=True), _PADV)
    xta = jnp.concatenate(
        [xtf.astype(jnp.bfloat16), xt2f.astype(jnp.bfloat16)], axis=1)

    m1 = m1_scr[...]
    m2 = m2_scr[...]
    for j in range(_KB // _L):
        v = jax.lax.dot_general(xa, xta[j * _L:(j + 1) * _L, :],
                                (((1,), (1,)), ((), ())),
                                preferred_element_type=jnp.float32)
        hi = jnp.maximum(m1, v)
        m1 = jnp.minimum(m1, v)
        m2 = jnp.minimum(m2, hi)
    m1_scr[...] = m1
    m2_scr[...] = m2

    @pl.when(pid == _NK - 1)
    def _epilogue():
        M1 = m1_scr[...]
        M2 = m2_scr[...]
        b1 = jnp.min(M1, axis=1, keepdims=True)
        eq = M1 == b1
        cnt = jnp.sum(eq.astype(jnp.float32), axis=1, keepdims=True)
        c_m1 = jnp.min(jnp.where(eq, _BIG, M1), axis=1, keepdims=True)
        c_m2 = jnp.min(jnp.where(eq, M2, _BIG), axis=1, keepdims=True)
        b2 = jnp.where(cnt > 1.0, b1, jnp.minimum(c_m1, c_m2))
        xs2 = xs2_scr[...]
        d0 = jnp.sqrt(jnp.maximum(b1 + xs2, 0.0))
        d1 = jnp.sqrt(jnp.maximum(b2 + xs2, 0.0))
        mem = (d0 < _THRESHOLD * d1).astype(jnp.float32)
        out_ref[...] = jnp.sum(mem, keepdims=True).reshape(1, 1) / jnp.float32(_Q)


def kernel(X_train, ts, x_init, zs, W):
    out = pl.pallas_call(
        _knn_kernel,
        grid=(_NK,),
        in_specs=[
            pl.BlockSpec(memory_space=pltpu.SMEM),
            pl.BlockSpec((_Q, _D), lambda k: (0, 0)),
            pl.BlockSpec((_Q, _T - 1, _D), lambda k: (0, 0, 0)),
            pl.BlockSpec((_D, _D), lambda k: (0, 0)),
            pl.BlockSpec((_KB, _D), lambda k: (k, 0)),
        ],
        out_specs=pl.BlockSpec((1, 1), lambda k: (0, 0)),
        out_shape=jax.ShapeDtypeStruct((1, 1), jnp.float32),
        scratch_shapes=[
            pltpu.VMEM((_Q, _D + 1), jnp.bfloat16),
            pltpu.VMEM((_Q, 1), jnp.float32),
            pltpu.VMEM((_Q, _L), jnp.float32),
            pltpu.VMEM((_Q, _L), jnp.float32),
        ],
        compiler_params=pltpu.CompilerParams(
            dimension_semantics=("arbitrary",),
        ),
    )(ts, x_init, zs, W, X_train)
    return out[0, 0]


# speedup vs baseline: 9.3621x; 1.0548x over previous
import jax
import jax.numpy as jnp
from jax.experimental import pallas as pl
from jax.experimental.pallas import tpu as pltpu

_Q = 1024
_K = 100000
_D = 32
_T = 10
_THRESHOLD = 1.0 / 3.0
_KB = 4096
_NK = (_K + _KB - 1) // _KB
_BIG = 3.0e38
_PADV = 1.0e30
_L = 128


def _knn_kernel(ts_ref, xinit_ref, zs_ref, w_ref, xt_ref, out_ref,
                xa_scr, xs2_scr, m1_scr, m2_scr):
    pid = pl.program_id(0)

    @pl.when(pid == 0)
    def _prologue():
        x = xinit_ref[...]
        w = w_ref[...]
        for i in range(_T - 1):
            dt = ts_ref[i + 1] - ts_ref[i]
            x = (x + jnp.dot(x, w, preferred_element_type=jnp.float32) * dt
                 + jnp.sqrt(jnp.abs(dt)) * zs_ref[:, i, :])
        xs2_scr[...] = jnp.sum(x * x, axis=1, keepdims=True)
        xa_scr[...] = jnp.concatenate(
            [-2.0 * x, jnp.ones((_Q, 1), jnp.float32)],
            axis=1).astype(jnp.bfloat16)
        m1_scr[...] = jnp.full((_Q, _L), _BIG, jnp.float32)
        m2_scr[...] = jnp.full((_Q, _L), _BIG, jnp.float32)

    xa = xa_scr[...]
    row = jax.lax.broadcasted_iota(jnp.int32, (_KB, 1), 0) + pid * _KB
    rmask = row < _K
    xtf = jnp.where(rmask, xt_ref[...], 0.0)
    xt2f = jnp.where(rmask,
                     jnp.sum(xtf * xtf, axis=1, keepdims=True), _PADV)
    xta = jnp.concatenate(
        [xtf.astype(jnp.bfloat16), xt2f.astype(jnp.bfloat16)], axis=1)

    m1 = m1_scr[...]
    m2 = m2_scr[...]
    for j in range(_KB // _L):
        v = jax.lax.dot_general(xa, xta[j * _L:(j + 1) * _L, :],
                                (((1,), (1,)), ((), ())),
                                preferred_element_type=jnp.float32)
        hi = jnp.maximum(m1, v)
        m1 = jnp.minimum(m1, v)
        m2 = jnp.minimum(m2, hi)
    m1_scr[...] = m1
    m2_scr[...] = m2

    @pl.when(pid == _NK - 1)
    def _epilogue():
        M1 = m1_scr[...]
        M2 = m2_scr[...]
        b1 = jnp.min(M1, axis=1, keepdims=True)
        eq = M1 == b1
        cnt = jnp.sum(eq.astype(jnp.float32), axis=1, keepdims=True)
        c_m1 = jnp.min(jnp.where(eq, _BIG, M1), axis=1, keepdims=True)
        c_m2 = jnp.min(jnp.where(eq, M2, _BIG), axis=1, keepdims=True)
        b2 = jnp.where(cnt > 1.0, b1, jnp.minimum(c_m1, c_m2))
        xs2 = xs2_scr[...]
        d0 = jnp.sqrt(jnp.maximum(b1 + xs2, 0.0))
        d1 = jnp.sqrt(jnp.maximum(b2 + xs2, 0.0))
        mem = (d0 < _THRESHOLD * d1).astype(jnp.float32)
        out_ref[...] = jnp.sum(mem, keepdims=True).reshape(1, 1) / jnp.float32(_Q)


def kernel(X_train, ts, x_init, zs, W):
    out = pl.pallas_call(
        _knn_kernel,
        grid=(_NK,),
        in_specs=[
            pl.BlockSpec(memory_space=pltpu.SMEM),
            pl.BlockSpec((_Q, _D), lambda k: (0, 0)),
            pl.BlockSpec((_Q, _T - 1, _D), lambda k: (0, 0, 0)),
            pl.BlockSpec((_D, _D), lambda k: (0, 0)),
            pl.BlockSpec((_KB, _D), lambda k: (k, 0)),
        ],
        out_specs=pl.BlockSpec((1, 1), lambda k: (0, 0)),
        out_shape=jax.ShapeDtypeStruct((1, 1), jnp.float32),
        scratch_shapes=[
            pltpu.VMEM((_Q, _D + 1), jnp.bfloat16),
            pltpu.VMEM((_Q, 1), jnp.float32),
            pltpu.VMEM((_Q, _L), jnp.float32),
            pltpu.VMEM((_Q, _L), jnp.float32),
        ],
        compiler_params=pltpu.CompilerParams(
            dimension_semantics=("arbitrary",),
        ),
    )(ts, x_init, zs, W, X_train)
    return out[0, 0]


# speedup vs baseline: 9.3810x; 1.0020x over previous
import jax
import jax.numpy as jnp
from jax.experimental import pallas as pl
from jax.experimental.pallas import tpu as pltpu

_Q = 1024
_K = 100000
_D = 32
_T = 10
_THRESHOLD = 1.0 / 3.0
_KB = 8192
_NK = (_K + _KB - 1) // _KB
_BIG = 3.0e38
_PADV = 1.0e30
_L = 128


def _knn_kernel(ts_ref, xinit_ref, zs_ref, w_ref, xt_ref, out_ref,
                xa_scr, xs2_scr, m1_scr, m2_scr):
    pid = pl.program_id(0)

    @pl.when(pid == 0)
    def _prologue():
        x = xinit_ref[...]
        w = w_ref[...]
        for i in range(_T - 1):
            dt = ts_ref[i + 1] - ts_ref[i]
            x = (x + jnp.dot(x, w, preferred_element_type=jnp.float32) * dt
                 + jnp.sqrt(jnp.abs(dt)) * zs_ref[:, i, :])
        xs2_scr[...] = jnp.sum(x * x, axis=1, keepdims=True)
        xa_scr[...] = jnp.concatenate(
            [-2.0 * x, jnp.ones((_Q, 1), jnp.float32)],
            axis=1).astype(jnp.bfloat16)
        m1_scr[...] = jnp.full((_Q, _L), _BIG, jnp.float32)
        m2_scr[...] = jnp.full((_Q, _L), _BIG, jnp.float32)

    xa = xa_scr[...]
    row = jax.lax.broadcasted_iota(jnp.int32, (_KB, 1), 0) + pid * _KB
    rmask = row < _K
    xtf = jnp.where(rmask, xt_ref[...], 0.0)
    xt2f = jnp.where(rmask,
                     jnp.sum(xtf * xtf, axis=1, keepdims=True), _PADV)
    xta = jnp.concatenate(
        [xtf.astype(jnp.bfloat16), xt2f.astype(jnp.bfloat16)], axis=1)

    m1 = m1_scr[...]
    m2 = m2_scr[...]
    for j in range(_KB // _L):
        v = jax.lax.dot_general(xa, xta[j * _L:(j + 1) * _L, :],
                                (((1,), (1,)), ((), ())),
                                preferred_element_type=jnp.float32)
        hi = jnp.maximum(m1, v)
        m1 = jnp.minimum(m1, v)
        m2 = jnp.minimum(m2, hi)
    m1_scr[...] = m1
    m2_scr[...] = m2

    @pl.when(pid == _NK - 1)
    def _epilogue():
        M1 = m1_scr[...]
        M2 = m2_scr[...]
        b1 = jnp.min(M1, axis=1, keepdims=True)
        eq = M1 == b1
        cnt = jnp.sum(eq.astype(jnp.float32), axis=1, keepdims=True)
        c_m1 = jnp.min(jnp.where(eq, _BIG, M1), axis=1, keepdims=True)
        c_m2 = jnp.min(jnp.where(eq, M2, _BIG), axis=1, keepdims=True)
        b2 = jnp.where(cnt > 1.0, b1, jnp.minimum(c_m1, c_m2))
        xs2 = xs2_scr[...]
        d0 = jnp.sqrt(jnp.maximum(b1 + xs2, 0.0))
        d1 = jnp.sqrt(jnp.maximum(b2 + xs2, 0.0))
        mem = (d0 < _THRESHOLD * d1).astype(jnp.float32)
        out_ref[...] = jnp.sum(mem, keepdims=True).reshape(1, 1) / jnp.float32(_Q)


def kernel(X_train, ts, x_init, zs, W):
    out = pl.pallas_call(
        _knn_kernel,
        grid=(_NK,),
        in_specs=[
            pl.BlockSpec(memory_space=pltpu.SMEM),
            pl.BlockSpec((_Q, _D), lambda k: (0, 0)),
            pl.BlockSpec((_Q, _T - 1, _D), lambda k: (0, 0, 0)),
            pl.BlockSpec((_D, _D), lambda k: (0, 0)),
            pl.BlockSpec((_KB, _D), lambda k: (k, 0)),
        ],
        out_specs=pl.BlockSpec((1, 1), lambda k: (0, 0)),
        out_shape=jax.ShapeDtypeStruct((1, 1), jnp.float32),
        scratch_shapes=[
            pltpu.VMEM((_Q, _D + 1), jnp.bfloat16),
            pltpu.VMEM((_Q, 1), jnp.float32),
            pltpu.VMEM((_Q, _L), jnp.float32),
            pltpu.VMEM((_Q, _L), jnp.float32),
        ],
        compiler_params=pltpu.CompilerParams(
            dimension_semantics=("arbitrary",),
        ),
    )(ts, x_init, zs, W, X_train)
    return out[0, 0]


# speedup vs baseline: 10.1801x; 1.0852x over previous
import jax
import jax.numpy as jnp
from jax.experimental import pallas as pl
from jax.experimental.pallas import tpu as pltpu

_Q = 1024
_K = 100000
_D = 32
_T = 10
_THRESHOLD = 1.0 / 3.0
_KB = 8192
_NK = (_K + _KB - 1) // _KB
_BIG = 3.0e38
_PADV = 1.0e30
_L = 128


def _knn_kernel(ts_ref, xinit_ref, zs_ref, w_ref, xt_ref, out_ref,
                xa_scr, xs2_scr, m1_scr, m2_scr):
    pid = pl.program_id(0)

    @pl.when(pid == 0)
    def _prologue():
        x = xinit_ref[...]
        w = w_ref[...]
        for i in range(_T - 1):
            dt = ts_ref[i + 1] - ts_ref[i]
            x = (x + jnp.dot(x, w, preferred_element_type=jnp.float32) * dt
                 + jnp.sqrt(jnp.abs(dt)) * zs_ref[:, i * _D:(i + 1) * _D])
        xs2_scr[...] = jnp.sum(x * x, axis=1, keepdims=True)
        xa_scr[...] = jnp.concatenate(
            [-2.0 * x, jnp.ones((_Q, 1), jnp.float32)],
            axis=1).astype(jnp.bfloat16)
        m1_scr[...] = jnp.full((_Q, _L), _BIG, jnp.float32)
        m2_scr[...] = jnp.full((_Q, _L), _BIG, jnp.float32)

    xa = xa_scr[...]
    row = jax.lax.broadcasted_iota(jnp.int32, (_KB, 1), 0) + pid * _KB
    rmask = row < _K
    xtf = jnp.where(rmask, xt_ref[...], 0.0)
    xt2f = jnp.where(rmask,
                     jnp.sum(xtf * xtf, axis=1, keepdims=True), _PADV)
    xta = jnp.concatenate(
        [xtf.astype(jnp.bfloat16), xt2f.astype(jnp.bfloat16)], axis=1)

    m1 = m1_scr[...]
    m2 = m2_scr[...]
    for j in range(_KB // _L):
        v = jax.lax.dot_general(xa, xta[j * _L:(j + 1) * _L, :],
                                (((1,), (1,)), ((), ())),
                                preferred_element_type=jnp.float32)
        hi = jnp.maximum(m1, v)
        m1 = jnp.minimum(m1, v)
        m2 = jnp.minimum(m2, hi)
    m1_scr[...] = m1
    m2_scr[...] = m2

    @pl.when(pid == _NK - 1)
    def _epilogue():
        M1 = m1_scr[...]
        M2 = m2_scr[...]
        b1 = jnp.min(M1, axis=1, keepdims=True)
        eq = M1 == b1
        cnt = jnp.sum(eq.astype(jnp.float32), axis=1, keepdims=True)
        c_m1 = jnp.min(jnp.where(eq, _BIG, M1), axis=1, keepdims=True)
        c_m2 = jnp.min(jnp.where(eq, M2, _BIG), axis=1, keepdims=True)
        b2 = jnp.where(cnt > 1.0, b1, jnp.minimum(c_m1, c_m2))
        xs2 = xs2_scr[...]
        d0 = jnp.sqrt(jnp.maximum(b1 + xs2, 0.0))
        d1 = jnp.sqrt(jnp.maximum(b2 + xs2, 0.0))
        mem = (d0 < _THRESHOLD * d1).astype(jnp.float32)
        out_ref[...] = jnp.sum(mem, keepdims=True).reshape(1, 1) / jnp.float32(_Q)


def kernel(X_train, ts, x_init, zs, W):
    out = pl.pallas_call(
        _knn_kernel,
        grid=(_NK,),
        in_specs=[
            pl.BlockSpec(memory_space=pltpu.SMEM),
            pl.BlockSpec((_Q, _D), lambda k: (0, 0)),
            pl.BlockSpec((_Q, (_T - 1) * _D), lambda k: (0, 0)),
            pl.BlockSpec((_D, _D), lambda k: (0, 0)),
            pl.BlockSpec((_KB, _D), lambda k: (k, 0)),
        ],
        out_specs=pl.BlockSpec((1, 1), lambda k: (0, 0)),
        out_shape=jax.ShapeDtypeStruct((1, 1), jnp.float32),
        scratch_shapes=[
            pltpu.VMEM((_Q, _D + 1), jnp.bfloat16),
            pltpu.VMEM((_Q, 1), jnp.float32),
            pltpu.VMEM((_Q, _L), jnp.float32),
            pltpu.VMEM((_Q, _L), jnp.float32),
        ],
        compiler_params=pltpu.CompilerParams(
            dimension_semantics=("arbitrary",),
        ),
    )(ts, x_init, zs.reshape(_Q, (_T - 1) * _D), W, X_train)
    return out[0, 0]
